# baseline (speedup 1.0000x reference)
import functools

import jax
import jax.numpy as jnp
from jax import lax
from jax.experimental import pallas as pl
from jax.experimental.pallas import tpu as pltpu
from jax.experimental.pallas import tpu_sc as plsc

_N_FIELDS = 26
_VOCAB = 100000
_DIM = 64
_BATCH = 16384

_NC = 2
_NS = 16
_NW = _NC * _NS

_B_TOTAL = _N_FIELDS * _BATCH
_R = _B_TOTAL // _NW
_C = 128
_NCHUNK = _R // _C
_T_ROWS = _N_FIELDS * _VOCAB

_mesh = plsc.VectorSubcoreMesh(core_axis_name="c", subcore_axis_name="s")


@functools.partial(
    pl.kernel,
    mesh=_mesh,
    compiler_params=pltpu.CompilerParams(
        use_tc_tiling_on_sc=True, needs_layout_passes=False
    ),
    out_type=jax.ShapeDtypeStruct((_B_TOTAL, _DIM), jnp.float32),
    scratch_types=[
        pltpu.VMEM((_NCHUNK, _C), jnp.int32),
        pltpu.SemaphoreType.DMA,
    ],
)
def _sc_gather(x_hbm, tf_hbm, out_hbm, idx_v, sem):
    wid = lax.axis_index("s") * _NC + lax.axis_index("c")
    base_row = wid * _R

    pltpu.sync_copy(x_hbm.at[wid], idx_v)

    def _preprocess(c, carry):
        off = ((base_row + c * _C) // _BATCH) * _VOCAB
        for s in range(_C // 16):
            sl = pl.ds(s * 16, 16)
            idx_v[c, sl] = idx_v[c, sl] + off
        return carry

    lax.fori_loop(0, _NCHUNK, _preprocess, 0)

    def _group(s, carry):
        v = idx_v[s >> 3, pl.ds((s & 7) * 16, 16)]
        j0 = base_row + s * 16
        for k in range(16):
            pltpu.async_copy(
                tf_hbm.at[pl.ds(v[k], 1)],
                out_hbm.at[pl.ds(j0 + k, 1)],
                sem,
            )
        return carry

    lax.fori_loop(0, _R // 16, _group, 0)

    pltpu.make_async_copy(
        tf_hbm.at[pl.ds(0, _R)],
        out_hbm.at[pl.ds(base_row, _R)],
        sem,
    ).wait()


def kernel(X, tables):
    xr = X.reshape(_NW, _NCHUNK, _C)
    tf = tables.reshape(_T_ROWS, _DIM)
    return _sc_gather(xr, tf)


# speedup vs baseline: 2.8807x; 2.8807x over previous
import functools

import jax
import jax.numpy as jnp
from jax import lax
from jax.experimental import pallas as pl
from jax.experimental.pallas import tpu as pltpu
from jax.experimental.pallas import tpu_sc as plsc

_N_FIELDS = 26
_VOCAB = 100000
_DIM = 64
_BATCH = 16384

_NC = 2
_NS = 16
_NW = _NC * _NS

_C = 128

_mesh = plsc.VectorSubcoreMesh(core_axis_name="c", subcore_axis_name="s")


def _make_gather(n_fields):
    b_total = n_fields * _BATCH
    rows = b_total // _NW
    nchunk = rows // _C

    @functools.partial(
        pl.kernel,
        mesh=_mesh,
        compiler_params=pltpu.CompilerParams(
            use_tc_tiling_on_sc=True, needs_layout_passes=False
        ),
        out_type=jax.ShapeDtypeStruct((b_total, 2 * _DIM), jnp.float32),
        scratch_types=[
            pltpu.VMEM((nchunk, _C), jnp.int32),
            pltpu.VMEM((6 * _C, 2 * _DIM), jnp.float32),
            pltpu.SemaphoreType.DMA,
            pltpu.SemaphoreType.DMA,
        ],
    )
    def _sc_gather(x_hbm, tp_hbm, out_hbm, idx_v, gbuf, gsem, osem):
        wid = lax.axis_index("s") * _NC + lax.axis_index("c")
        base_row = wid * rows

        pltpu.sync_copy(x_hbm.at[wid], idx_v)

        def _preprocess(c, carry):
            off = ((base_row + c * _C) // _BATCH) * _VOCAB
            for s in range(_C // 16):
                sl = pl.ds(s * 16, 16)
                idx_v[c, sl] = idx_v[c, sl] + off
            return carry

        lax.fori_loop(0, nchunk, _preprocess, 0)

        def _fire_gather(c, b):
            pltpu.async_copy(
                tp_hbm.at[idx_v.at[c]], gbuf.at[pl.ds(b * _C, _C)], gsem
            )

        def _drain_gather(b):
            pltpu.make_async_copy(
                tp_hbm.at[pl.ds(0, _C)], gbuf.at[pl.ds(b * _C, _C)], gsem
            ).wait()

        def _fire_writeback(c, b):
            pltpu.async_copy(
                gbuf.at[pl.ds(b * _C, _C)],
                out_hbm.at[pl.ds(base_row + c * _C, _C)],
                osem,
            )

        def _drain_writeback(b):
            pltpu.make_async_copy(
                tp_hbm.at[pl.ds(0, _C)],
                gbuf.at[pl.ds(b * _C, _C)],
                osem,
            ).wait()

        for c in range(4):
            _fire_gather(c, c)

        for w in (0, 1):
            _fire_gather(w + 4, (w + 4) % 6)
            _drain_gather(w % 6)
            _fire_writeback(w, w % 6)

        def _chunk(w, carry):
            _drain_writeback((w - 2) % 6)
            _fire_gather(w + 4, (w + 4) % 6)
            _drain_gather(w % 6)
            _fire_writeback(w, w % 6)
            return carry

        lax.fori_loop(2, nchunk - 4, _chunk, 0)

        for w in range(nchunk - 4, nchunk):
            _drain_writeback((w - 2) % 6)
            _drain_gather(w % 6)
            _fire_writeback(w, w % 6)

        _drain_writeback((nchunk - 2) % 6)
        _drain_writeback((nchunk - 1) % 6)

    return _sc_gather


_HALF = _N_FIELDS // 2
_gather_half = _make_gather(_HALF)


def _half_call(X_half, tables_half):
    xr = X_half.reshape(_NW, (_HALF * _BATCH) // (_NW * _C), _C)
    tp = jnp.pad(tables_half, ((0, 0), (0, 0), (0, _DIM))).reshape(
        _HALF * _VOCAB, 2 * _DIM
    )
    return _gather_half(xr, tp)


def kernel(X, tables):
    o1 = _half_call(X[:_HALF], tables[:_HALF])
    o2 = _half_call(X[_HALF:], tables[_HALF:])
    return jnp.concatenate([o1, o2], axis=0)[:, :_DIM]


# speedup vs baseline: 4.6612x; 1.6181x over previous
import functools

import jax
import jax.numpy as jnp
from jax import lax
from jax.experimental import pallas as pl
from jax.experimental.pallas import tpu as pltpu
from jax.experimental.pallas import tpu_sc as plsc

_N_FIELDS = 26
_VOCAB = 100000
_DIM = 64
_BATCH = 16384

_NC = 2
_NS = 16
_NW = _NC * _NS

_B_TOTAL = _N_FIELDS * _BATCH
_R = _B_TOTAL // _NW
_C = 128
_NCHUNK = _R // _C
_T_ROWS = _N_FIELDS * _VOCAB

_mesh = plsc.VectorSubcoreMesh(core_axis_name="c", subcore_axis_name="s")


@functools.partial(
    pl.kernel,
    mesh=_mesh,
    compiler_params=pltpu.CompilerParams(
        use_tc_tiling_on_sc=True, needs_layout_passes=False
    ),
    out_type=jax.ShapeDtypeStruct((_B_TOTAL, 2 * _DIM), jnp.float32),
    scratch_types=[
        pltpu.VMEM((_NCHUNK, _C), jnp.int32),
        pltpu.VMEM((6 * _C, 2 * _DIM), jnp.float32),
        pltpu.SemaphoreType.DMA,
        pltpu.SemaphoreType.DMA,
    ],
)
def _sc_gather(x_hbm, tp_hbm, out_hbm, idx_v, gbuf, gsem, osem):
    wid = lax.axis_index("s") * _NC + lax.axis_index("c")
    base_row = wid * _R

    pltpu.sync_copy(x_hbm.at[wid], idx_v)

    def _preprocess(c, carry):
        off = ((base_row + c * _C) // _BATCH) * _VOCAB
        for s in range(_C // 16):
            sl = pl.ds(s * 16, 16)
            idx_v[c, sl] = idx_v[c, sl] + off
        return carry

    lax.fori_loop(0, _NCHUNK, _preprocess, 0)

    def _fire_gather(c, b):
        pltpu.async_copy(tp_hbm.at[idx_v.at[c]], gbuf.at[pl.ds(b * _C, _C)], gsem)

    def _drain_gather(b):
        pltpu.make_async_copy(
            tp_hbm.at[pl.ds(0, _C)], gbuf.at[pl.ds(b * _C, _C)], gsem
        ).wait()

    def _fire_writeback(c, b):
        pltpu.async_copy(
            gbuf.at[pl.ds(b * _C, _C)],
            out_hbm.at[pl.ds(base_row + c * _C, _C)],
            osem,
        )

    def _drain_writeback(b):
        pltpu.make_async_copy(
            tp_hbm.at[pl.ds(0, _C)],
            gbuf.at[pl.ds(b * _C, _C)],
            osem,
        ).wait()

    for c in range(4):
        _fire_gather(c, c)

    for w in (0, 1):
        _fire_gather(w + 4, (w + 4) % 6)
        _drain_gather(w % 6)
        _fire_writeback(w, w % 6)

    def _chunk(w, carry):
        _drain_writeback((w - 2) % 6)
        _fire_gather(w + 4, (w + 4) % 6)
        _drain_gather(w % 6)
        _fire_writeback(w, w % 6)
        return carry

    lax.fori_loop(2, _NCHUNK - 4, _chunk, 0)

    for w in range(_NCHUNK - 4, _NCHUNK):
        _drain_writeback((w - 2) % 6)
        _drain_gather(w % 6)
        _fire_writeback(w, w % 6)

    _drain_writeback((_NCHUNK - 2) % 6)
    _drain_writeback((_NCHUNK - 1) % 6)


def kernel(X, tables):
    xr = X.reshape(_NW, _NCHUNK, _C)
    tp = jnp.pad(tables, ((0, 0), (0, 0), (0, _DIM))).reshape(_T_ROWS, 2 * _DIM)
    padded_out = _sc_gather(xr, tp)
    return padded_out[:, :_DIM]
